# baseline (speedup 1.0000x reference)
import functools

import jax
import jax.numpy as jnp
from jax import lax
from jax.experimental import pallas as pl
from jax.experimental.pallas import tpu as pltpu
from jax.experimental.pallas import tpu_sc as plsc

_NC = 2
_NS = 16
_L = 16
_NW = _NC * _NS
_ROW = 128


@functools.partial(jax.jit, static_argnums=(4, 5))
def _sc_mask(edge_index, gids, starts2d, zeros_hbm, n_nodes, n_edges):
    epw = n_edges // _NW
    npc = n_nodes // _NS
    nrows = npc // _ROW
    mesh = plsc.VectorSubcoreMesh(core_axis_name="c", subcore_axis_name="s")

    @functools.partial(
        pl.kernel,
        mesh=mesh,
        compiler_params=pltpu.CompilerParams(needs_layout_passes=False),
        out_type=(jax.ShapeDtypeStruct((n_edges // 2,), jnp.int32),
                  jax.ShapeDtypeStruct((n_edges // 2,), jnp.int32)),
        scratch_types=[
            pltpu.VMEM((n_nodes,), jnp.int32),
            pltpu.VMEM((n_nodes,), jnp.int32),
            pltpu.VMEM((epw,), jnp.int32),
            pltpu.VMEM((epw,), jnp.int32),
            pltpu.VMEM((epw,), jnp.int32),
            pltpu.VMEM((nrows, _ROW), jnp.int32),
            pltpu.VMEM((_ROW,), jnp.int32),
            pltpu.VMEM_SHARED((n_nodes,), jnp.int32),
            pltpu.VMEM_SHARED((n_nodes,), jnp.int32),
            pltpu.SemaphoreType.DMA,
            pltpu.SemaphoreType.DMA,
            pltpu.SemaphoreType.DMA,
        ],
    )
    def k(edge_hbm, gids_hbm, starts_hbm, z_hbm, out0_hbm, out1_hbm,
          start_tab, gid_tab, h_v, t_v, o_v, idx2, ones_v,
          start_sp, gid_sp, sem, sem2, sem3):
        cid = lax.axis_index("c")
        sid = lax.axis_index("s")
        obase = sid * epw
        base = cid * (n_edges // 2) + obase

        cp_h = pltpu.make_async_copy(edge_hbm.at[0, pl.ds(base, epw)], h_v, sem)
        cp_h.start()
        cp_t = pltpu.make_async_copy(edge_hbm.at[1, pl.ds(base, epw)], t_v, sem)
        cp_t.start()

        soff = sid * npc
        st0 = pltpu.async_copy(z_hbm.at[pl.ds(soff, npc)],
                               start_sp.at[pl.ds(soff, npc)], sem2)
        st1 = pltpu.async_copy(gids_hbm.at[pl.ds(soff, npc)],
                               gid_sp.at[pl.ds(soff, npc)], sem2)
        st2 = pltpu.async_copy(starts_hbm.at[pl.ds(sid * nrows, nrows), :],
                               idx2, sem2)
        ones = jnp.full((_L,), 1, jnp.int32)
        for j in range(_ROW // _L):
            ones_v[pl.ds(j * _L, _L)] = ones
        st0.wait()
        st1.wait()
        st2.wait()

        plsc.subcore_barrier()
        gb = pltpu.make_async_copy(gid_sp, gid_tab, sem3)
        gb.start()
        scats = [
            pltpu.async_copy(ones_v, start_sp.at[idx2.at[j]], sem2, add=True)
            for j in range(nrows)
        ]
        for s in scats:
            s.wait()
        plsc.subcore_barrier()

        sb = pltpu.make_async_copy(start_sp, start_tab, sem2)
        sb.start()
        cp_h.wait()
        cp_t.wait()
        gb.wait()
        sb.wait()

        zeros = jnp.zeros((_L,), jnp.int32)
        onesl = jnp.full((_L,), 1, jnp.int32)
        neg1 = jnp.full((_L,), -1, jnp.int32)

        @plsc.parallel_loop(0, epw // _L, step=1, unroll=8)
        def edge_body(i):
            off = i * _L
            h = h_v[pl.ds(off, _L)]
            t = t_v[pl.ds(off, _L)]
            s = plsc.load_gather(start_tab, [h])
            g = plsc.load_gather(gid_tab, [t])
            m = (s != zeros) & (g != neg1)
            o_v[pl.ds(off, _L)] = jnp.where(m, onesl, zeros)

        @pl.when(cid == 0)
        def _():
            pltpu.sync_copy(o_v, out0_hbm.at[pl.ds(obase, epw)])

        @pl.when(cid == 1)
        def _():
            pltpu.sync_copy(o_v, out1_hbm.at[pl.ds(obase, epw)])

    out0, out1 = k(edge_index, gids, starts2d, zeros_hbm)
    return jnp.concatenate([out0, out1])


def kernel(edge_index, edge_batch, node_global_ids, node_ptr, edge_ptr,
           start_node_locals, start_ptr, start_entity_ids, start_entity_ptr,
           answer_node_locals, answer_ptr, answer_entity_ids,
           edge_relations, edge_labels, is_answer_reachable):
    n_edges = edge_index.shape[1]
    n_nodes = node_global_ids.shape[0]
    ei = edge_index.astype(jnp.int32)
    gids = node_global_ids.astype(jnp.int32)
    starts2d = start_node_locals.astype(jnp.int32).reshape(-1, _ROW)
    zeros_hbm = jnp.zeros((n_nodes,), jnp.int32)
    out = _sc_mask(ei, gids, starts2d, zeros_hbm, n_nodes, n_edges)
    return out != 0


# speedup vs baseline: 1.0012x; 1.0012x over previous
import functools

import jax
import jax.numpy as jnp
from jax import lax
from jax.experimental import pallas as pl
from jax.experimental.pallas import tpu as pltpu
from jax.experimental.pallas import tpu_sc as plsc

_NC = 2
_NS = 16
_L = 16
_NW = _NC * _NS
_ROW = 128


@functools.partial(jax.jit, static_argnums=(4, 5))
def _sc_mask(edge_index, gids, starts2d, zeros_hbm, n_nodes, n_edges):
    epw = n_edges // _NW
    npc = n_nodes // _NS
    nrows = npc // _ROW
    mesh = plsc.VectorSubcoreMesh(core_axis_name="c", subcore_axis_name="s")

    @functools.partial(
        pl.kernel,
        mesh=mesh,
        compiler_params=pltpu.CompilerParams(needs_layout_passes=False),
        out_type=jax.ShapeDtypeStruct((n_edges,), jnp.int32),
        scratch_types=[
            pltpu.VMEM((n_nodes,), jnp.int32),
            pltpu.VMEM((n_nodes,), jnp.int32),
            pltpu.VMEM((epw,), jnp.int32),
            pltpu.VMEM((epw,), jnp.int32),
            pltpu.VMEM((epw,), jnp.int32),
            pltpu.VMEM((nrows, _ROW), jnp.int32),
            pltpu.VMEM((_ROW,), jnp.int32),
            pltpu.VMEM_SHARED((n_nodes,), jnp.int32),
            pltpu.VMEM_SHARED((n_nodes,), jnp.int32),
            pltpu.SemaphoreType.DMA,
            pltpu.SemaphoreType.DMA,
            pltpu.SemaphoreType.DMA,
        ],
    )
    def k(edge_hbm, gids_hbm, starts_hbm, z_hbm, out_hbm,
          start_tab, gid_tab, h_v, t_v, o_v, idx2, ones_v,
          start_sp, gid_sp, sem, sem2, sem3):
        cid = lax.axis_index("c")
        sid = lax.axis_index("s")
        wid = sid * _NC + cid
        base = wid * epw

        cp_h = pltpu.make_async_copy(edge_hbm.at[0, pl.ds(base, epw)], h_v, sem)
        cp_h.start()
        cp_t = pltpu.make_async_copy(edge_hbm.at[1, pl.ds(base, epw)], t_v, sem)
        cp_t.start()

        soff = sid * npc
        st0 = pltpu.async_copy(z_hbm.at[pl.ds(soff, npc)],
                               start_sp.at[pl.ds(soff, npc)], sem2)
        st1 = pltpu.async_copy(gids_hbm.at[pl.ds(soff, npc)],
                               gid_sp.at[pl.ds(soff, npc)], sem2)
        st2 = pltpu.async_copy(starts_hbm.at[pl.ds(sid * nrows, nrows), :],
                               idx2, sem2)
        ones = jnp.full((_L,), 1, jnp.int32)
        for j in range(_ROW // _L):
            ones_v[pl.ds(j * _L, _L)] = ones
        st0.wait()
        st1.wait()
        st2.wait()

        plsc.subcore_barrier()
        gb = pltpu.make_async_copy(gid_sp, gid_tab, sem3)
        gb.start()
        scats = [
            pltpu.async_copy(ones_v, start_sp.at[idx2.at[j]], sem2, add=True)
            for j in range(nrows)
        ]
        for s in scats:
            s.wait()
        plsc.subcore_barrier()

        sb = pltpu.make_async_copy(start_sp, start_tab, sem2)
        sb.start()
        cp_h.wait()
        cp_t.wait()
        gb.wait()
        sb.wait()

        zeros = jnp.zeros((_L,), jnp.int32)
        onesl = jnp.full((_L,), 1, jnp.int32)
        neg1 = jnp.full((_L,), -1, jnp.int32)

        @plsc.parallel_loop(0, epw // _L, step=1, unroll=8)
        def edge_body(i):
            off = i * _L
            h = h_v[pl.ds(off, _L)]
            t = t_v[pl.ds(off, _L)]
            s = plsc.load_gather(start_tab, [h])
            g = plsc.load_gather(gid_tab, [t])
            m = (s != zeros) & (g != neg1)
            o_v[pl.ds(off, _L)] = jnp.where(m, onesl, zeros)

        pltpu.sync_copy(o_v, out_hbm.at[pl.ds(base, epw)])

    return k(edge_index, gids, starts2d, zeros_hbm)


def kernel(edge_index, edge_batch, node_global_ids, node_ptr, edge_ptr,
           start_node_locals, start_ptr, start_entity_ids, start_entity_ptr,
           answer_node_locals, answer_ptr, answer_entity_ids,
           edge_relations, edge_labels, is_answer_reachable):
    n_edges = edge_index.shape[1]
    n_nodes = node_global_ids.shape[0]
    ei = edge_index.astype(jnp.int32)
    gids = node_global_ids.astype(jnp.int32)
    starts2d = start_node_locals.astype(jnp.int32).reshape(-1, _ROW)
    zeros_hbm = jnp.zeros((n_nodes,), jnp.int32)
    out = _sc_mask(ei, gids, starts2d, zeros_hbm, n_nodes, n_edges)
    return out != 0
